# baseline (speedup 1.0000x reference)
import functools

import jax
import jax.numpy as jnp
from jax import lax
from jax.experimental import pallas as pl
from jax.experimental.pallas import tpu as pltpu
from jax.experimental.pallas import tpu_sc as plsc

D = 128
S = 9
O = 128

NC = 2
NS = 16
L = 16
NW = NC * NS

CH = 32
NCHUNK = 50
PER_W = CH * NCHUNK
NPAD = NW * PER_W
ROWS = CH * S
GR = 96
NG = ROWS // GR
IDX_W = PER_W * S

BR = 25088
NBLK = 2
NP = NBLK * BR


def _matmul_body(x_ref, w_ref, z_ref):
    z_ref[...] = jnp.dot(x_ref[...], w_ref[...],
                         preferred_element_type=jnp.float32)


def _tc_matmul(x, wcat):
    return pl.pallas_call(
        _matmul_body,
        grid=(NBLK, S),
        in_specs=[
            pl.BlockSpec((BR, D), lambda i, s: (i, 0)),
            pl.BlockSpec((D, O), lambda i, s: (0, s)),
        ],
        out_specs=pl.BlockSpec((BR, O), lambda i, s: (s * NBLK + i, 0)),
        out_shape=jax.ShapeDtypeStruct((S * NP, O), jnp.float32),
    )(x, wcat)


def _sc_body(z_ref, idx_ref, b_ref, out_ref,
             fbuf, rawbuf, pattbuf, binit, gb0, gb1, shA, shB, bbuf,
             semA, semB, osemA, osemB):
    sid = lax.axis_index("s")
    wid = sid * NC + lax.axis_index("c")
    base = wid * PER_W
    sid_base = sid * CH
    bufA = (gb0, shA, semA, osemA)
    bufB = (gb1, shB, semB, osemB)

    pltpu.sync_copy(b_ref, bbuf)
    bvecs = [bbuf[pl.ds(p * L, L)] for p in range(O // L)]

    for n in range(CH):
        for p in range(O // L):
            binit[n, pl.ds(p * L, L)] = bvecs[p]

    for s9 in range(S):
        for h in range(CH // L):
            jv = lax.iota(jnp.int32, L) + h * L
            pattbuf[s9, pl.ds(h * L, L)] = jv + sid_base

    pltpu.sync_copy(idx_ref.at[pl.ds(base * S, IDX_W)], rawbuf)

    def flat_body(k, carry):
        for s9 in range(S):
            for h in range(CH // L):
                pos = (lax.iota(jnp.int32, L) + h * L) * S + (k * ROWS + s9)
                rawv = plsc.load_gather(rawbuf, [pos])
                fbuf[pl.ds(k * ROWS + s9 * CH + h * L, L)] = rawv + s9 * NP
        return carry

    lax.fori_loop(0, NCHUNK, flat_body, 0)

    def fire(k, buf):
        gbuf = buf[0]
        for g in range(NG):
            pltpu.async_copy(
                z_ref.at[fbuf.at[pl.ds(k * ROWS + g * GR, GR)]],
                gbuf.at[pl.ds(g * GR, GR)], buf[2])

    def drain_acc_store(k, buf):
        gbuf, sh, sem, osem = buf
        acc = sh.at[pl.ds(sid_base, CH)]

        @pl.when(k >= 2)
        def _():
            pltpu.make_async_copy(acc, out_ref.at[pl.ds(base, CH)],
                                  osem).wait()
        pltpu.sync_copy(binit, acc)

        for g in range(NG):
            pltpu.make_async_copy(
                z_ref.at[fbuf.at[pl.ds(k * ROWS + g * GR, GR)]],
                gbuf.at[pl.ds(g * GR, GR)], sem).wait()

        for s9 in range(S):
            pltpu.sync_copy(gbuf.at[pl.ds(s9 * CH, CH)],
                            sh.at[pattbuf.at[s9]], add=True)
        pltpu.async_copy(acc, out_ref.at[pl.ds(base + k * CH, CH)], osem)

    fire(0, bufA)

    def pair_body(t, carry):
        k0 = 2 * t
        fire(k0 + 1, bufB)
        drain_acc_store(k0, bufA)

        @pl.when(k0 + 2 < NCHUNK)
        def _():
            fire(k0 + 2, bufA)

        drain_acc_store(k0 + 1, bufB)
        return carry

    lax.fori_loop(0, NCHUNK // 2, pair_body, 0)

    pltpu.make_async_copy(shA.at[pl.ds(sid_base, CH)],
                          out_ref.at[pl.ds(base, CH)], osemA).wait()
    pltpu.make_async_copy(shB.at[pl.ds(sid_base, CH)],
                          out_ref.at[pl.ds(base, CH)], osemB).wait()


_sc_gather_sum = functools.partial(
    pl.kernel,
    out_type=jax.ShapeDtypeStruct((NPAD, O), jnp.float32),
    mesh=plsc.VectorSubcoreMesh(core_axis_name="c", subcore_axis_name="s",
                                num_cores=NC, num_subcores=NS),
    compiler_params=pltpu.CompilerParams(needs_layout_passes=False),
    scratch_types=(
        [pltpu.VMEM((IDX_W,), jnp.int32),
         pltpu.VMEM((IDX_W,), jnp.int32),
         pltpu.VMEM((S, CH), jnp.int32),
         pltpu.VMEM((CH, O), jnp.float32)]
        + [pltpu.VMEM((ROWS, O), jnp.float32)] * 2
        + [pltpu.VMEM_SHARED((NS * CH, O), jnp.float32)] * 2
        + [pltpu.VMEM((O,), jnp.float32),
           pltpu.SemaphoreType.DMA,
           pltpu.SemaphoreType.DMA,
           pltpu.SemaphoreType.DMA,
           pltpu.SemaphoreType.DMA]
    ),
)(_sc_body)


def kernel(x, indices, W, b):
    n_nodes = x.shape[0]
    wcat = W.reshape(S, D, O).transpose(1, 0, 2).reshape(D, S * O)
    z = _tc_matmul(x.astype(jnp.bfloat16), wcat.astype(jnp.bfloat16))
    idx_pad = jnp.pad(indices, ((0, NPAD - n_nodes), (0, 0))).reshape(-1)
    out = _sc_gather_sum(z, idx_pad.astype(jnp.int32), b)
    return out[:n_nodes]
